# Initial kernel scaffold; baseline (speedup 1.0000x reference)
import functools

import jax
import jax.numpy as jnp
from jax import lax
from jax.experimental import pallas as pl
from jax.experimental.pallas import tpu as pltpu
from jax.experimental.pallas import tpu_sc as plsc

NC = 2
NS = 16
NW = NC * NS

SUB = 128
K = 8
CHUNK = K * SUB


def _make_emb_kernel(B, D):
    assert B % (NW * CHUNK) == 0
    b_per_w = B // NW
    n_chunks = b_per_w // CHUNK
    rows_per_w = b_per_w // SUB

    mesh = plsc.VectorSubcoreMesh(core_axis_name="c", subcore_axis_name="s")

    @functools.partial(
        pl.kernel,
        mesh=mesh,
        out_type=jax.ShapeDtypeStruct((B, D), jnp.float32),
        scratch_types=[
            pltpu.VMEM((K, SUB), jnp.int32),
            pltpu.VMEM((CHUNK, D), jnp.float32),
            pltpu.SemaphoreType.DMA,
        ],
    )
    def emb(idx_hbm, table_hbm, out_hbm, idx_v, rows_v, sem):
        wid = lax.axis_index("s") * NC + lax.axis_index("c")
        row_base = wid * rows_per_w

        def chunk_body(i, _):
            irow = row_base + i * K
            pltpu.sync_copy(idx_hbm.at[pl.ds(irow, K)], idx_v)
            copies = []
            for j in range(K):
                copies.append(
                    pltpu.async_copy(
                        table_hbm.at[idx_v.at[j]],
                        rows_v.at[pl.ds(j * SUB, SUB)],
                        sem,
                    )
                )
            for c in copies:
                c.wait()
            off = irow * SUB
            pltpu.sync_copy(rows_v, out_hbm.at[pl.ds(off, CHUNK)])
            return ()

        lax.fori_loop(0, n_chunks, chunk_body, ())

    return emb


def kernel(indices, table):
    BATCH, HIST = indices.shape
    V, D = table.shape
    B = BATCH * HIST
    idx2d = indices.reshape(B // SUB, SUB).astype(jnp.int32)
    emb = _make_emb_kernel(B, D)
    out = emb(idx2d, table)
    return out.reshape(BATCH, HIST, D)


# speedup vs baseline: 4.3629x; 4.3629x over previous
import functools

import jax
import jax.numpy as jnp
from jax import lax
from jax.experimental import pallas as pl
from jax.experimental.pallas import tpu as pltpu
from jax.experimental.pallas import tpu_sc as plsc

NC = 2
NS = 16
NW = NC * NS

SUB = 128
K = 8
CHUNK = K * SUB


def _make_emb_kernel(B, DP):
    assert B % (NW * CHUNK) == 0
    b_per_w = B // NW
    n_chunks = b_per_w // CHUNK

    mesh = plsc.VectorSubcoreMesh(core_axis_name="c", subcore_axis_name="s")

    @functools.partial(
        pl.kernel,
        mesh=mesh,
        out_type=jax.ShapeDtypeStruct((B, DP), jnp.float32),
        scratch_types=(
            [pltpu.VMEM((SUB,), jnp.int32) for _ in range(K)]
            + [
                pltpu.VMEM((CHUNK, DP), jnp.float32),
                pltpu.SemaphoreType.DMA,
                pltpu.SemaphoreType.DMA,
            ]
        ),
        compiler_params=pltpu.CompilerParams(use_tc_tiling_on_sc=False),
    )
    def emb(idx_hbm, table_hbm, out_hbm, *rest):
        idx_bufs = rest[:K]
        rows_v = rest[K]
        sem_i = rest[K + 1]
        sem_g = rest[K + 2]
        wid = lax.axis_index("s") * NC + lax.axis_index("c")
        base = wid * b_per_w

        def chunk_body(i, _):
            off = base + i * CHUNK
            icopies = [
                pltpu.async_copy(
                    idx_hbm.at[pl.ds(off + j * SUB, SUB)], idx_bufs[j], sem_i
                )
                for j in range(K)
            ]
            for c in icopies:
                c.wait()
            gcopies = [
                pltpu.async_copy(
                    table_hbm.at[idx_bufs[j]],
                    rows_v.at[pl.ds(j * SUB, SUB)],
                    sem_g,
                )
                for j in range(K)
            ]
            for c in gcopies:
                c.wait()
            pltpu.sync_copy(rows_v, out_hbm.at[pl.ds(off, CHUNK)])
            return ()

        lax.fori_loop(0, n_chunks, chunk_body, ())

    return emb


def kernel(indices, table):
    BATCH, HIST = indices.shape
    V, D = table.shape
    DP = (D + 7) // 8 * 8
    B = BATCH * HIST
    idx_flat = indices.reshape(B).astype(jnp.int32)
    table_p = jnp.pad(table, ((0, 0), (0, DP - D)))
    emb = _make_emb_kernel(B, DP)
    out = emb(idx_flat, table_p)
    return out[:, :D].reshape(BATCH, HIST, D)
